# baseline (speedup 1.0000x reference)
import dataclasses
import functools

import jax
import jax.numpy as jnp
from jax import lax
from jax.experimental import pallas as pl
from jax.experimental.pallas import tpu as pltpu
from jax.experimental.pallas import tpu_sc as plsc

WH = 1025
NH = 16
NV = 3972
NVP = 3976
BR = 8
BC = 128
GR = 128
GC = 8
LANES = 16


def _compiler_params():
    cp = pltpu.CompilerParams()
    if "needs_layout_passes" in pltpu.CompilerParams.__dataclass_fields__:
        cp = dataclasses.replace(cp, needs_layout_passes=False)
    return cp


def _bias_gather(table_flat, idx):
    mesh = plsc.VectorSubcoreMesh(core_axis_name="c", subcore_axis_name="s")

    @functools.partial(
        pl.kernel,
        mesh=mesh,
        out_type=jax.ShapeDtypeStruct((NH, WH, WH), jnp.float32),
        compiler_params=_compiler_params(),
        scratch_types=[pltpu.VMEM((NVP,), jnp.float32) for _ in range(NH)],
    )
    def k(tab_hbm, idx_hbm, out_hbm, *tab_refs):
        for h in range(NH):
            pltpu.sync_copy(tab_hbm.at[pl.ds(h * NVP, NVP)], tab_refs[h])

        def body(idx_v, out_v):
            jj = pl.program_id(1)

            @plsc.parallel_loop(0, BR * (BC // LANES), unroll=2)
            def _vreg(g):
                r = g // (BC // LANES)
                cv = g % (BC // LANES)
                iv = idx_v[r, pl.ds(jj * BC + cv * LANES, LANES)]
                for h in range(NH):
                    out_v[h, r, pl.ds(cv * LANES, LANES)] = plsc.load_gather(
                        tab_refs[h], [iv]
                    )

        pltpu.emit_pipeline(
            body,
            grid=(GR, GC),
            in_specs=[pl.BlockSpec((BR, GC * BC), index_map=lambda i, j: (i, 0))],
            out_specs=[pl.BlockSpec((NH, BR, BC), index_map=lambda i, j: (0, i, j))],
            core_axis_name=("c", "s"),
            dimension_semantics=(pltpu.PARALLEL, pltpu.PARALLEL),
        )(
            idx_hbm.at[pl.ds(0, GR * BR), pl.ds(0, GC * BC)],
            out_hbm.at[:, pl.ds(0, GR * BR), pl.ds(0, GC * BC)],
        )

    return k(table_flat, idx)


def kernel(relative_position_bias_table, relative_position_index):
    table_t = relative_position_bias_table.T
    table_flat = jnp.pad(table_t, ((0, 0), (0, NVP - NV))).reshape(-1)
    idx = relative_position_index.astype(jnp.int32)
    out = _bias_gather(table_flat, idx)
    row_vals = jnp.take(relative_position_bias_table, idx[WH - 1, :], axis=0)
    col_vals = jnp.take(relative_position_bias_table, idx[:, WH - 1], axis=0)
    out = lax.dynamic_update_slice(out, row_vals.T.reshape(NH, 1, WH), (0, WH - 1, 0))
    out = lax.dynamic_update_slice(out, col_vals.T.reshape(NH, WH, 1), (0, 0, WH - 1))
    return out


# speedup vs baseline: 1.0810x; 1.0810x over previous
import dataclasses
import functools

import jax
import jax.numpy as jnp
from jax import lax
from jax.experimental import pallas as pl
from jax.experimental.pallas import tpu as pltpu
from jax.experimental.pallas import tpu_sc as plsc

WH = 1025
NH = 16
NV = 3972
NVP = 3976
EP = 1032
BR = 8
BC = 128
GR = 128
GC = 8
LANES = 16


def _compiler_params():
    cp = pltpu.CompilerParams()
    if "needs_layout_passes" in pltpu.CompilerParams.__dataclass_fields__:
        cp = dataclasses.replace(cp, needs_layout_passes=False)
    return cp


def _bias_gather(table_flat, idx, ecol, erow):
    mesh = plsc.VectorSubcoreMesh(core_axis_name="c", subcore_axis_name="s")

    @functools.partial(
        pl.kernel,
        mesh=mesh,
        out_type=jax.ShapeDtypeStruct((NH, WH, WH), jnp.float32),
        compiler_params=_compiler_params(),
        scratch_types=[pltpu.VMEM((NVP,), jnp.float32) for _ in range(NH)]
        + [
            pltpu.VMEM((NH, 1, BC), jnp.float32),
            pltpu.VMEM((NH, BR, 1), jnp.float32),
            pltpu.VMEM((NH, 1, 1), jnp.float32),
            pltpu.VMEM((EP,), jnp.int32),
            pltpu.VMEM((EP,), jnp.int32),
        ],
    )
    def k(tab_hbm, idx_hbm, ecol_hbm, erow_hbm, out_hbm, *refs):
        tab_refs = refs[:NH]
        rowbuf, colbuf, cornerbuf, ecol_v, erow_v = refs[NH:]
        wid = lax.axis_index("s") * 2 + lax.axis_index("c")
        for h in range(NH):
            pltpu.sync_copy(tab_hbm.at[pl.ds(h * NVP, NVP)], tab_refs[h])

        def body(idx_v, out_v):
            @plsc.parallel_loop(0, BR * (BC // LANES), unroll=2)
            def _vreg(g):
                r = g // (BC // LANES)
                cv = g % (BC // LANES)
                iv = idx_v[r, pl.ds(cv * LANES, LANES)]
                for h in range(NH):
                    out_v[h, r, pl.ds(cv * LANES, LANES)] = plsc.load_gather(
                        tab_refs[h], [iv]
                    )

        pltpu.emit_pipeline(
            body,
            grid=(GR, GC),
            in_specs=[pl.BlockSpec((BR, BC), index_map=lambda i, j: (i, j))],
            out_specs=[pl.BlockSpec((NH, BR, BC), index_map=lambda i, j: (0, i, j))],
            core_axis_name=("c", "s"),
            dimension_semantics=(pltpu.PARALLEL, pltpu.PARALLEL),
        )(
            idx_hbm.at[pl.ds(0, GR * BR), pl.ds(0, GC * BC)],
            out_hbm.at[:, pl.ds(0, GR * BR), pl.ds(0, GC * BC)],
        )

        pltpu.sync_copy(ecol_hbm, ecol_v)
        pltpu.sync_copy(erow_hbm, erow_v)
        iota = lax.iota(jnp.int32, LANES)

        @pl.when(wid < 8)
        def _row_edge():
            c0 = pl.multiple_of(wid * BC, BC)
            for cv in range(BC // LANES):
                iv = erow_v[pl.ds(wid * BC + cv * LANES, LANES)]
                for h in range(NH):
                    rowbuf[h, 0, pl.ds(cv * LANES, LANES)] = plsc.load_gather(
                        tab_refs[h], [iv]
                    )
            pltpu.sync_copy(
                rowbuf,
                out_hbm.at[:, pl.ds(GR * BR, 1), pl.ds(c0, BC)],
            )

        for t in range(4):
            r0 = pl.multiple_of((wid * 4 + t) * BR, BR)
            iv = ecol_v[pl.ds((wid * 4 + t) * BR, LANES)]
            for h in range(NH):
                vh = plsc.load_gather(tab_refs[h], [iv])
                plsc.store_scatter(
                    colbuf,
                    [jnp.full((LANES,), h, jnp.int32), iota, jnp.zeros((LANES,), jnp.int32)],
                    vh,
                    mask=iota < BR,
                )
            pltpu.sync_copy(
                colbuf,
                out_hbm.at[:, pl.ds(r0, BR), pl.ds(GC * BC, 1)],
            )

        @pl.when(wid == 8)
        def _corner():
            iv = ecol_v[pl.ds(1016, LANES)]
            for h in range(NH):
                vh = plsc.load_gather(tab_refs[h], [iv])
                plsc.store_scatter(
                    cornerbuf,
                    [jnp.full((LANES,), h, jnp.int32), iota - 8, iota - 8],
                    vh,
                    mask=iota == 8,
                )
            pltpu.sync_copy(
                cornerbuf,
                out_hbm.at[:, pl.ds(GR * BR, 1), pl.ds(GC * BC, 1)],
            )

    return k(table_flat, idx, ecol, erow)


def kernel(relative_position_bias_table, relative_position_index):
    table_t = relative_position_bias_table.T
    table_flat = jnp.pad(table_t, ((0, 0), (0, NVP - NV))).reshape(-1)
    idx = relative_position_index.astype(jnp.int32)
    ecol = jnp.pad(idx[:, WH - 1], (0, EP - WH))
    erow = jnp.pad(idx[WH - 1, :], (0, EP - WH))
    return _bias_gather(table_flat, idx, ecol, erow)


# speedup vs baseline: 1.1125x; 1.0291x over previous
import dataclasses
import functools

import jax
import jax.numpy as jnp
from jax import lax
from jax.experimental import pallas as pl
from jax.experimental.pallas import tpu as pltpu
from jax.experimental.pallas import tpu_sc as plsc

WH = 1025
NH = 16
NHC = 8
NV = 3972
NVP = 3976
EP = 1032
BR = 8
BC = 512
GR = 128
GC = 2
LANES = 16


def _compiler_params():
    cp = pltpu.CompilerParams()
    if "needs_layout_passes" in pltpu.CompilerParams.__dataclass_fields__:
        cp = dataclasses.replace(cp, needs_layout_passes=False)
    return cp


def _bias_gather(table_flat, idx, ecol, erow):
    mesh = plsc.VectorSubcoreMesh(core_axis_name="c", subcore_axis_name="s")

    @functools.partial(
        pl.kernel,
        mesh=mesh,
        out_type=jax.ShapeDtypeStruct((NH, WH, WH), jnp.float32),
        compiler_params=_compiler_params(),
        scratch_types=[pltpu.VMEM((NVP,), jnp.float32) for _ in range(NHC)]
        + [
            pltpu.VMEM((NHC, 1, 128), jnp.float32),
            pltpu.VMEM((NHC, 8, 1), jnp.float32),
            pltpu.VMEM((NHC, 1, 1), jnp.float32),
            pltpu.VMEM((EP,), jnp.int32),
            pltpu.VMEM((EP,), jnp.int32),
        ],
    )
    def k(tab_hbm, idx_hbm, ecol_hbm, erow_hbm, out_hbm, *refs):
        tab_refs = refs[:NHC]
        rowbuf, colbuf, cornerbuf, ecol_v, erow_v = refs[NHC:]
        c = lax.axis_index("c")
        s = lax.axis_index("s")
        h0 = c * NHC
        for h in range(NHC):
            pltpu.sync_copy(tab_hbm.at[pl.ds((h0 + h) * NVP, NVP)], tab_refs[h])

        out_c = out_hbm.at[pl.ds(h0, NHC)]

        def body(idx_v, out_v):
            @plsc.parallel_loop(0, BR * (BC // LANES), unroll=2)
            def _vreg(g):
                r = g // (BC // LANES)
                cv = g % (BC // LANES)
                iv = idx_v[r, pl.ds(cv * LANES, LANES)]
                for h in range(NHC):
                    out_v[h, r, pl.ds(cv * LANES, LANES)] = plsc.load_gather(
                        tab_refs[h], [iv]
                    )

        pltpu.emit_pipeline(
            body,
            grid=(GR, GC),
            in_specs=[pl.BlockSpec((BR, BC), index_map=lambda i, j: (i, j))],
            out_specs=[pl.BlockSpec((NHC, BR, BC), index_map=lambda i, j: (0, i, j))],
            core_axis_name=("s",),
            dimension_semantics=(pltpu.PARALLEL, pltpu.PARALLEL),
        )(
            idx_hbm.at[pl.ds(0, GR * BR), pl.ds(0, GC * BC)],
            out_c.at[:, pl.ds(0, GR * BR), pl.ds(0, GC * BC)],
        )

        pltpu.sync_copy(ecol_hbm, ecol_v)
        pltpu.sync_copy(erow_hbm, erow_v)
        iota = lax.iota(jnp.int32, LANES)

        @pl.when(s < 8)
        def _row_edge():
            c0 = pl.multiple_of(s * 128, 128)
            for cv in range(128 // LANES):
                iv = erow_v[pl.ds(s * 128 + cv * LANES, LANES)]
                for h in range(NHC):
                    rowbuf[h, 0, pl.ds(cv * LANES, LANES)] = plsc.load_gather(
                        tab_refs[h], [iv]
                    )
            pltpu.sync_copy(
                rowbuf,
                out_c.at[:, pl.ds(GR * BR, 1), pl.ds(c0, 128)],
            )

        for t in range(8):
            r0 = pl.multiple_of((s * 8 + t) * 8, 8)
            iv = ecol_v[pl.ds((s * 8 + t) * 8, LANES)]
            for h in range(NHC):
                vh = plsc.load_gather(tab_refs[h], [iv])
                plsc.store_scatter(
                    colbuf,
                    [jnp.full((LANES,), h, jnp.int32), iota, jnp.zeros((LANES,), jnp.int32)],
                    vh,
                    mask=iota < 8,
                )
            pltpu.sync_copy(
                colbuf,
                out_c.at[:, pl.ds(r0, 8), pl.ds(GC * BC, 1)],
            )

        @pl.when(s == 8)
        def _corner():
            iv = ecol_v[pl.ds(1016, LANES)]
            for h in range(NHC):
                vh = plsc.load_gather(tab_refs[h], [iv])
                plsc.store_scatter(
                    cornerbuf,
                    [jnp.full((LANES,), h, jnp.int32), iota - 8, iota - 8],
                    vh,
                    mask=iota == 8,
                )
            pltpu.sync_copy(
                cornerbuf,
                out_c.at[:, pl.ds(GR * BR, 1), pl.ds(GC * BC, 1)],
            )

    return k(table_flat, idx, ecol, erow)


def kernel(relative_position_bias_table, relative_position_index):
    table_t = relative_position_bias_table.T
    table_flat = jnp.pad(table_t, ((0, 0), (0, NVP - NV))).reshape(-1)
    idx = relative_position_index.astype(jnp.int32)
    ecol = jnp.pad(idx[:, WH - 1], (0, EP - WH))
    erow = jnp.pad(idx[WH - 1, :], (0, EP - WH))
    return _bias_gather(table_flat, idx, ecol, erow)


# speedup vs baseline: 1.1620x; 1.0445x over previous
import dataclasses
import functools

import jax
import jax.numpy as jnp
from jax import lax
from jax.experimental import pallas as pl
from jax.experimental.pallas import tpu as pltpu
from jax.experimental.pallas import tpu_sc as plsc

WH = 1025
NH = 16
NHC = 8
NV = 3972
NVP = 3976
EP = 1032
BR = 8
BC = 512
GR = 128
GC = 2
LANES = 16


def _compiler_params():
    cp = pltpu.CompilerParams()
    if "needs_layout_passes" in pltpu.CompilerParams.__dataclass_fields__:
        cp = dataclasses.replace(cp, needs_layout_passes=False)
    return cp


def _bias_gather(table_flat, idx, ecol, erow):
    mesh = plsc.VectorSubcoreMesh(core_axis_name="c", subcore_axis_name="s")

    @functools.partial(
        pl.kernel,
        mesh=mesh,
        out_type=jax.ShapeDtypeStruct((NH, WH, WH), jnp.float32),
        compiler_params=_compiler_params(),
        scratch_types=[pltpu.VMEM((NVP,), jnp.float32) for _ in range(NHC)]
        + [
            pltpu.VMEM((NHC, 1, 128), jnp.float32),
            pltpu.VMEM((NHC, 8, 1), jnp.float32),
            pltpu.VMEM((NHC, 1, 1), jnp.float32),
            pltpu.VMEM((EP,), jnp.int32),
            pltpu.VMEM((EP,), jnp.int32),
        ],
    )
    def k(tab_hbm, idx_hbm, ecol_hbm, erow_hbm, out_hbm, *refs):
        tab_refs = refs[:NHC]
        rowbuf, colbuf, cornerbuf, ecol_v, erow_v = refs[NHC:]
        c = lax.axis_index("c")
        s = lax.axis_index("s")
        h0 = c * NHC
        for h in range(NHC):
            pltpu.sync_copy(tab_hbm.at[pl.ds((h0 + h) * NVP, NVP)], tab_refs[h])

        out_c = out_hbm.at[pl.ds(h0, NHC)]

        def body(idx_v, out_v):
            @plsc.parallel_loop(0, BR * (BC // LANES), unroll=4)
            def _vreg(g):
                r = g // (BC // LANES)
                cv = g % (BC // LANES)
                iv = idx_v[r, pl.ds(cv * LANES, LANES)]
                for h in range(NHC):
                    out_v[h, r, pl.ds(cv * LANES, LANES)] = plsc.load_gather(
                        tab_refs[h], [iv]
                    )

        pltpu.emit_pipeline(
            body,
            grid=(GR, GC),
            in_specs=[pl.BlockSpec((BR, BC), index_map=lambda i, j: (i, j))],
            out_specs=[pl.BlockSpec((NHC, BR, BC), index_map=lambda i, j: (0, i, j))],
            core_axis_name=("s",),
            dimension_semantics=(pltpu.PARALLEL, pltpu.PARALLEL),
        )(
            idx_hbm.at[pl.ds(0, GR * BR), pl.ds(0, GC * BC)],
            out_c.at[:, pl.ds(0, GR * BR), pl.ds(0, GC * BC)],
        )

        pltpu.sync_copy(ecol_hbm, ecol_v)
        pltpu.sync_copy(erow_hbm, erow_v)
        iota = lax.iota(jnp.int32, LANES)

        @pl.when(s < 8)
        def _row_edge():
            c0 = pl.multiple_of(s * 128, 128)
            for cv in range(128 // LANES):
                iv = erow_v[pl.ds(s * 128 + cv * LANES, LANES)]
                for h in range(NHC):
                    rowbuf[h, 0, pl.ds(cv * LANES, LANES)] = plsc.load_gather(
                        tab_refs[h], [iv]
                    )
            pltpu.sync_copy(
                rowbuf,
                out_c.at[:, pl.ds(GR * BR, 1), pl.ds(c0, 128)],
            )

        for t in range(8):
            r0 = pl.multiple_of((s * 8 + t) * 8, 8)
            iv = ecol_v[pl.ds((s * 8 + t) * 8, LANES)]
            for h in range(NHC):
                vh = plsc.load_gather(tab_refs[h], [iv])
                plsc.store_scatter(
                    colbuf,
                    [jnp.full((LANES,), h, jnp.int32), iota, jnp.zeros((LANES,), jnp.int32)],
                    vh,
                    mask=iota < 8,
                )
            pltpu.sync_copy(
                colbuf,
                out_c.at[:, pl.ds(r0, 8), pl.ds(GC * BC, 1)],
            )

        @pl.when(s == 8)
        def _corner():
            iv = ecol_v[pl.ds(1016, LANES)]
            for h in range(NHC):
                vh = plsc.load_gather(tab_refs[h], [iv])
                plsc.store_scatter(
                    cornerbuf,
                    [jnp.full((LANES,), h, jnp.int32), iota - 8, iota - 8],
                    vh,
                    mask=iota == 8,
                )
            pltpu.sync_copy(
                cornerbuf,
                out_c.at[:, pl.ds(GR * BR, 1), pl.ds(GC * BC, 1)],
            )

    return k(table_flat, idx, ecol, erow)


def kernel(relative_position_bias_table, relative_position_index):
    table_t = relative_position_bias_table.T
    table_flat = jnp.pad(table_t, ((0, 0), (0, NVP - NV))).reshape(-1)
    idx = relative_position_index.astype(jnp.int32)
    ecol = jnp.pad(idx[:, WH - 1], (0, EP - WH))
    erow = jnp.pad(idx[WH - 1, :], (0, EP - WH))
    return _bias_gather(table_flat, idx, ecol, erow)
